# baseline (speedup 1.0000x reference)
import functools

import jax
import jax.numpy as jnp
from jax import lax
from jax.experimental import pallas as pl
from jax.experimental.pallas import tpu as pltpu
from jax.experimental.pallas import tpu_sc as plsc

NC = 2
NS = 16
L = 16
NW = NC * NS

R = 16
NXB = 3
NEB = 2


def kernel(x, emb):
    batch, seq_len, d = x.shape
    max_len = emb.shape[0]
    spw = seq_len // NW
    nch = spw // R
    ch = R * d

    xf = x.reshape(batch * seq_len * d)
    ef = emb.reshape(max_len * d)

    mesh = plsc.VectorSubcoreMesh(core_axis_name="c", subcore_axis_name="s")

    @functools.partial(
        pl.kernel,
        out_type=jax.ShapeDtypeStruct((batch * seq_len * d,), x.dtype),
        mesh=mesh,
        scratch_types=[pltpu.VMEM((ch,), jnp.float32)] * (NXB + NEB)
        + [pltpu.SemaphoreType.DMA] * (NXB + NEB + NXB),
    )
    def sc_add(x_hbm, emb_hbm, out_hbm, *scratch):
        xbufs = scratch[:NXB]
        ebufs = scratch[NXB:NXB + NEB]
        sems = scratch[NXB + NEB:]
        xsems = sems[:NXB]
        esems = sems[NXB:NXB + NEB]
        osems = sems[NXB + NEB:]

        wid = lax.axis_index("s") * NC + lax.axis_index("c")
        s_base = wid * spw

        steps = [(c, b) for c in range(nch) for b in range(batch)]
        K = len(steps)

        def x_slice(k):
            c, b = steps[k]
            off = (b * seq_len + s_base + c * R) * d
            return pl.ds(off, ch)

        def start_x(k):
            return pltpu.async_copy(
                x_hbm.at[x_slice(k)], xbufs[k % NXB], xsems[k % NXB])

        def start_e(c):
            off = (s_base + c * R) * d
            return pltpu.async_copy(
                emb_hbm.at[pl.ds(off, ch)], ebufs[c % NEB], esems[c % NEB])

        edesc = [None] * nch
        xdesc = [None] * K
        odesc = [None] * K

        edesc[0] = start_e(0)
        xdesc[0] = start_x(0)

        for k, (c, b) in enumerate(steps):
            if b == 0 and c + 1 < nch:
                edesc[c + 1] = start_e(c + 1)
            if k + 1 < K:
                prev = k + 1 - NXB
                if prev >= 0:
                    odesc[prev].wait()
                    odesc[prev] = None
                xdesc[k + 1] = start_x(k + 1)
            if b == 0:
                edesc[c].wait()
            xdesc[k].wait()

            xb = xbufs[k % NXB]
            eb = ebufs[c % NEB]

            @pl.loop(0, ch // L, unroll=8)
            def _add(j, xb=xb, eb=eb):
                sl = pl.ds(j * L, L)
                plsc.addupdate(xb.at[sl], eb[sl])

            odesc[k] = pltpu.async_copy(
                xb, out_hbm.at[x_slice(k)], osems[k % NXB])

        for k in range(K):
            if odesc[k] is not None:
                odesc[k].wait()

    out = sc_add(xf, ef)
    return out.reshape(batch, seq_len, d)


# speedup vs baseline: 1.6654x; 1.6654x over previous
import functools

import jax
import jax.numpy as jnp
from jax import lax
from jax.experimental import pallas as pl
from jax.experimental.pallas import tpu as pltpu
from jax.experimental.pallas import tpu_sc as plsc

NC = 2
NS = 16
L = 16
NW = NC * NS

R = 16
NXB = 3
NEB = 2


def kernel(x, emb):
    batch, seq_len, d = x.shape
    spw = seq_len // NW
    nch = spw // R
    hpr = d // L

    x2 = x.reshape(batch * seq_len, d)

    mesh = plsc.VectorSubcoreMesh(core_axis_name="c", subcore_axis_name="s")

    @functools.partial(
        pl.kernel,
        out_type=jax.ShapeDtypeStruct((batch * seq_len, d), x.dtype),
        mesh=mesh,
        scratch_types=[pltpu.VMEM((R, d), jnp.float32)] * (NXB + NEB)
        + [pltpu.SemaphoreType.DMA] * (NXB + NEB + NXB),
    )
    def sc_add(x_hbm, emb_hbm, out_hbm, *scratch):
        xbufs = scratch[:NXB]
        ebufs = scratch[NXB:NXB + NEB]
        sems = scratch[NXB + NEB:]
        xsems = sems[:NXB]
        esems = sems[NXB:NXB + NEB]
        osems = sems[NXB + NEB:]

        wid = lax.axis_index("s") * NC + lax.axis_index("c")
        s_base = wid * spw

        steps = [(c, b) for c in range(nch) for b in range(batch)]
        K = len(steps)

        def x_rows(k):
            c, b = steps[k]
            return pl.ds(b * seq_len + s_base + c * R, R)

        def start_x(k):
            return pltpu.async_copy(
                x_hbm.at[x_rows(k)], xbufs[k % NXB], xsems[k % NXB])

        def start_e(c):
            return pltpu.async_copy(
                emb_hbm.at[pl.ds(s_base + c * R, R)],
                ebufs[c % NEB], esems[c % NEB])

        edesc = [None] * nch
        xdesc = [None] * K
        odesc = [None] * K

        edesc[0] = start_e(0)
        xdesc[0] = start_x(0)

        for k, (c, b) in enumerate(steps):
            if b == 0 and c + 1 < nch:
                edesc[c + 1] = start_e(c + 1)
            if k + 1 < K:
                prev = k + 1 - NXB
                if prev >= 0:
                    odesc[prev].wait()
                    odesc[prev] = None
                xdesc[k + 1] = start_x(k + 1)
            if b == 0:
                edesc[c].wait()
            xdesc[k].wait()

            xb = xbufs[k % NXB]
            eb = ebufs[c % NEB]

            @pl.loop(0, R * hpr, unroll=8)
            def _add(j, xb=xb, eb=eb):
                r = j >> 6
                col = (j & (hpr - 1)) * L
                sl = pl.ds(col, L)
                plsc.addupdate(xb.at[r, sl], eb[r, sl])

            odesc[k] = pltpu.async_copy(
                xb, out_hbm.at[x_rows(k)], osems[k % NXB])

        for k in range(K):
            if odesc[k] is not None:
                odesc[k].wait()

    out = sc_add(x2, emb)
    return out.reshape(batch, seq_len, d)


# speedup vs baseline: 2.7587x; 1.6564x over previous
import functools

import jax
import jax.numpy as jnp
from jax import lax
from jax.experimental import pallas as pl
from jax.experimental.pallas import tpu as pltpu
from jax.experimental.pallas import tpu_sc as plsc

NC = 2
NS = 16
L = 16
NW = NC * NS

R = 16
NXB = 3
NEB = 2


def kernel(x, emb):
    batch, seq_len, d = x.shape
    spw = seq_len // NW
    nch = spw // R
    hpr = d // L

    x2 = x.reshape(batch * seq_len, d)

    mesh = plsc.VectorSubcoreMesh(core_axis_name="c", subcore_axis_name="s")

    @functools.partial(
        pl.kernel,
        out_type=jax.ShapeDtypeStruct((batch * seq_len, d), x.dtype),
        mesh=mesh,
        scratch_types=[pltpu.VMEM((R, d), jnp.float32)] * (NXB + NEB)
        + [pltpu.SemaphoreType.DMA] * (NXB + NEB + NXB),
    )
    def sc_add(x_hbm, emb_hbm, out_hbm, *scratch):
        xbufs = scratch[:NXB]
        ebufs = scratch[NXB:NXB + NEB]
        sems = scratch[NXB + NEB:]
        xsems = sems[:NXB]
        esems = sems[NXB:NXB + NEB]
        osems = sems[NXB + NEB:]

        wid = lax.axis_index("s") * NC + lax.axis_index("c")
        s_base = wid * spw

        steps = [(c, b) for c in range(nch) for b in range(batch)]
        K = len(steps)

        def x_rows(k):
            c, b = steps[k]
            return pl.ds(b * seq_len + s_base + c * R, R)

        def start_x(k):
            return pltpu.async_copy(
                x_hbm.at[x_rows(k)], xbufs[k % NXB], xsems[k % NXB])

        def start_e(c):
            return pltpu.async_copy(
                emb_hbm.at[pl.ds(s_base + c * R, R)],
                ebufs[c % NEB], esems[c % NEB])

        edesc = [None] * nch
        xdesc = [None] * K
        odesc = [None] * K

        edesc[0] = start_e(0)
        xdesc[0] = start_x(0)

        for k, (c, b) in enumerate(steps):
            if b == 0 and c + 1 < nch:
                edesc[c + 1] = start_e(c + 1)
            if k + 1 < K:
                prev = k + 1 - NXB
                if prev >= 0:
                    odesc[prev].wait()
                    odesc[prev] = None
                xdesc[k + 1] = start_x(k + 1)
            if b == 0:
                edesc[c].wait()
            xdesc[k].wait()

            xb = xbufs[k % NXB]
            eb = ebufs[c % NEB]

            @plsc.parallel_loop(0, R * hpr, unroll=16)
            def _add(j, xb=xb, eb=eb):
                r = j // hpr
                col = (j % hpr) * L
                sl = pl.ds(col, L)
                plsc.addupdate(xb.at[r, sl], eb[r, sl])

            odesc[k] = pltpu.async_copy(
                xb, out_hbm.at[x_rows(k)], osems[k % NXB])

        for k in range(K):
            if odesc[k] is not None:
                odesc[k].wait()

    out = sc_add(x2, emb)
    return out.reshape(batch, seq_len, d)


# speedup vs baseline: 2.9193x; 1.0582x over previous
import jax
import jax.numpy as jnp
from jax import lax
from jax.experimental import pallas as pl
from jax.experimental.pallas import tpu as pltpu
from jax.experimental.pallas import tpu_sc as plsc

NC = 2
NS = 16
L = 16
NW = NC * NS

R = 8
NSET = 2


def kernel(x, emb):
    batch, seq_len, d = x.shape
    spw = seq_len // NW
    nch = spw // R
    hpr = d // L

    x2 = x.reshape(batch * seq_len, d)

    mesh = plsc.VectorSubcoreMesh(core_axis_name="c", subcore_axis_name="s")

    @pl.kernel(
        out_type=jax.ShapeDtypeStruct((batch * seq_len, d), x.dtype),
        mesh=mesh,
        scratch_types=[pltpu.VMEM((R, d), jnp.float32)] * (NSET * (batch + 1))
        + [pltpu.SemaphoreType.DMA] * (2 * NSET),
    )
    def sc_add(x_hbm, emb_hbm, out_hbm, *scratch):
        nbuf = batch + 1
        xbufs = [scratch[s * nbuf:s * nbuf + batch] for s in range(NSET)]
        ebufs = [scratch[s * nbuf + batch] for s in range(NSET)]
        sems = scratch[NSET * nbuf:]
        isems = sems[:NSET]
        osems = sems[NSET:]

        wid = lax.axis_index("s") * NC + lax.axis_index("c")
        s_base = wid * spw

        def start_in(c):
            st = c % NSET
            s0 = s_base + c * R
            descs = [
                pltpu.async_copy(
                    x_hbm.at[pl.ds(b * seq_len + s0, R)],
                    xbufs[st][b], isems[st])
                for b in range(batch)
            ]
            descs.append(
                pltpu.async_copy(emb_hbm.at[pl.ds(s0, R)], ebufs[st],
                                 isems[st]))
            return descs

        def start_out(c):
            st = c % NSET
            s0 = s_base + c * R
            return [
                pltpu.async_copy(
                    xbufs[st][b],
                    out_hbm.at[pl.ds(b * seq_len + s0, R)], osems[st])
                for b in range(batch)
            ]

        in_descs = [None] * NSET
        out_descs = [None] * NSET
        in_descs[0] = start_in(0)

        for c in range(nch):
            st = c % NSET
            if c + 1 < nch:
                nx = (c + 1) % NSET
                if out_descs[nx] is not None:
                    for od in out_descs[nx]:
                        od.wait()
                in_descs[nx] = start_in(c + 1)
            for idd in in_descs[st]:
                idd.wait()

            xbs = xbufs[st]
            eb = ebufs[st]

            @plsc.parallel_loop(0, R * hpr, unroll=8)
            def _add(g, xbs=xbs, eb=eb):
                r = g // hpr
                sl = pl.ds((g % hpr) * L, L)
                v = eb[r, sl]
                for b in range(batch):
                    plsc.addupdate(xbs[b].at[r, sl], v)

            out_descs[st] = start_out(c)

        for st in range(NSET):
            if out_descs[st] is not None:
                for od in out_descs[st]:
                    od.wait()

    out = sc_add(x2, emb)
    return out.reshape(batch, seq_len, d)


# speedup vs baseline: 2.9318x; 1.0043x over previous
import jax
import jax.numpy as jnp
from jax import lax
from jax.experimental import pallas as pl
from jax.experimental.pallas import tpu as pltpu
from jax.experimental.pallas import tpu_sc as plsc

NC = 2
NS = 16
L = 16
NW = NC * NS

R = 8
NSET = 3


def kernel(x, emb):
    batch, seq_len, d = x.shape
    spw = seq_len // NW
    nch = spw // R
    hpr = d // L

    x2 = x.reshape(batch * seq_len, d)

    mesh = plsc.VectorSubcoreMesh(core_axis_name="c", subcore_axis_name="s")

    @pl.kernel(
        out_type=jax.ShapeDtypeStruct((batch * seq_len, d), x.dtype),
        mesh=mesh,
        scratch_types=[pltpu.VMEM((R, d), jnp.float32)] * (NSET * (batch + 1))
        + [pltpu.SemaphoreType.DMA] * (2 * NSET),
    )
    def sc_add(x_hbm, emb_hbm, out_hbm, *scratch):
        nbuf = batch + 1
        xbufs = [scratch[s * nbuf:s * nbuf + batch] for s in range(NSET)]
        ebufs = [scratch[s * nbuf + batch] for s in range(NSET)]
        sems = scratch[NSET * nbuf:]
        isems = sems[:NSET]
        osems = sems[NSET:]

        wid = lax.axis_index("s") * NC + lax.axis_index("c")
        s_base = wid * spw

        def start_in(c):
            st = c % NSET
            s0 = s_base + c * R
            descs = [
                pltpu.async_copy(
                    x_hbm.at[pl.ds(b * seq_len + s0, R)],
                    xbufs[st][b], isems[st])
                for b in range(batch)
            ]
            descs.append(
                pltpu.async_copy(emb_hbm.at[pl.ds(s0, R)], ebufs[st],
                                 isems[st]))
            return descs

        def start_out(c):
            st = c % NSET
            s0 = s_base + c * R
            return [
                pltpu.async_copy(
                    xbufs[st][b],
                    out_hbm.at[pl.ds(b * seq_len + s0, R)], osems[st])
                for b in range(batch)
            ]

        in_descs = [None] * NSET
        out_descs = [None] * NSET
        in_descs[0] = start_in(0)

        for c in range(nch):
            st = c % NSET
            if c + 1 < nch:
                nx = (c + 1) % NSET
                if out_descs[nx] is not None:
                    for od in out_descs[nx]:
                        od.wait()
                in_descs[nx] = start_in(c + 1)
            for idd in in_descs[st]:
                idd.wait()

            xbs = xbufs[st]
            eb = ebufs[st]

            @plsc.parallel_loop(0, R * hpr, unroll=8)
            def _add(g, xbs=xbs, eb=eb):
                r = g // hpr
                sl = pl.ds((g % hpr) * L, L)
                v = eb[r, sl]
                for b in range(batch):
                    plsc.addupdate(xbs[b].at[r, sl], v)

            out_descs[st] = start_out(c)

        for st in range(NSET):
            if out_descs[st] is not None:
                for od in out_descs[st]:
                    od.wait()

    out = sc_add(x2, emb)
    return out.reshape(batch, seq_len, d)
